# baseline (speedup 1.0000x reference)
import functools

import jax
import jax.numpy as jnp
from jax import lax
from jax.experimental import pallas as pl
from jax.experimental.pallas import tpu as pltpu
from jax.experimental.pallas import tpu_sc as plsc

_NC = 2
_NS = 16
_NT = _NC * _NS
_CHUNK = 128
_STRIPE = 128


def _sc_mesh():
    return plsc.VectorSubcoreMesh(core_axis_name="c", subcore_axis_name="s")


@functools.cache
def _make_deg_kernel(ch, npad):
    @functools.partial(
        pl.kernel,
        mesh=_sc_mesh(),
        compiler_params=pltpu.CompilerParams(needs_layout_passes=False, use_tc_tiling_on_sc=False),
        out_type=jax.ShapeDtypeStruct((_NT, npad), jnp.float32),
        scratch_types=[
            pltpu.VMEM((ch, _CHUNK), jnp.int32),
            pltpu.VMEM((npad,), jnp.float32),
        ],
    )
    def deg_k(dst_hbm, out_hbm, idx_v, deg_v):
        cid = lax.axis_index("c")
        sid = lax.axis_index("s")
        wid = cid * _NS + sid
        z16 = jnp.zeros((16,), jnp.float32)
        ones16 = jnp.ones((16,), jnp.float32)

        def zero_body(i, carry):
            deg_v[pl.ds(i * 16, 16)] = z16
            return carry

        lax.fori_loop(0, npad // 16, zero_body, 0)
        pltpu.sync_copy(dst_hbm.at[wid], idx_v)

        def acc_body(r, carry):
            for c in range(_CHUNK // 16):
                idx = idx_v[r, pl.ds(c * 16, 16)]
                plsc.addupdate_scatter(deg_v, [idx], ones16)
            return carry

        lax.fori_loop(0, ch, acc_body, 0)
        pltpu.sync_copy(deg_v, out_hbm.at[wid])

    return deg_k


@functools.cache
def _make_conv_kernel(ch, npad, w):
    rows_per_tile = npad // _NS
    nstripe = rows_per_tile // _STRIPE

    @functools.partial(
        pl.kernel,
        mesh=_sc_mesh(),
        compiler_params=pltpu.CompilerParams(needs_layout_passes=False, use_tc_tiling_on_sc=False),
        out_type=jax.ShapeDtypeStruct((npad, 128), jnp.float32),
        scratch_types=[
            pltpu.VMEM((ch, _CHUNK), jnp.int32),
            pltpu.VMEM((ch, _CHUNK), jnp.int32),
            pltpu.VMEM((2, _CHUNK, w), jnp.float32),
            pltpu.VMEM_SHARED((npad, w), jnp.float32),
            pltpu.VMEM_SHARED((npad, w), jnp.float32),
            pltpu.SemaphoreType.DMA,
            pltpu.SemaphoreType.DMA,
        ],
    )
    def conv_k(g_hbm, src_hbm, dst_hbm, out_hbm, src_v, dst_v, rows_v, acc_sh,
               g_sh, sem0, sem1):
        cid = lax.axis_index("c")
        sid = lax.axis_index("s")
        wid = cid * _NS + sid
        sems = (sem0, sem1)
        z16 = jnp.zeros((16,), jnp.float32)

        def zero_body(r, carry):
            for c in range(w // 16):
                rows_v[0, r, pl.ds(c * 16, 16)] = z16
            return carry

        lax.fori_loop(0, _CHUNK, zero_body, 0)
        base = sid * rows_per_tile
        for k in range(nstripe):
            s = pl.ds(base + k * _STRIPE, _STRIPE)
            pltpu.sync_copy(g_hbm.at[s], g_sh.at[s])
            pltpu.sync_copy(rows_v.at[0], acc_sh.at[s])
        pltpu.sync_copy(src_hbm.at[wid], src_v)
        pltpu.sync_copy(dst_hbm.at[wid], dst_v)
        plsc.subcore_barrier()

        def start_gather(j, b):
            pltpu.async_copy(g_sh.at[src_v.at[j]], rows_v.at[b], sems[b])

        def finish(j, b):
            pltpu.make_async_copy(
                g_sh.at[src_v.at[j]], rows_v.at[b], sems[b]
            ).wait()
            pltpu.sync_copy(rows_v.at[b], acc_sh.at[dst_v.at[j]], add=True)

        start_gather(0, 0)
        npairs = (ch - 1) // 2

        def pair_body(p, carry):
            j0 = 2 * p
            start_gather(j0 + 1, 1)
            finish(j0, 0)
            start_gather(j0 + 2, 0)
            finish(j0 + 1, 1)
            return carry

        lax.fori_loop(0, npairs, pair_body, 0)
        if ch % 2 == 1:
            finish(ch - 1, 0)
        else:
            start_gather(ch - 1, 1)
            finish(ch - 2, 0)
            finish(ch - 1, 1)
        plsc.subcore_barrier()
        for k in range(nstripe):
            s = pl.ds(base + k * _STRIPE, _STRIPE)
            pltpu.sync_copy(acc_sh.at[s], rows_v.at[0])
            pltpu.sync_copy(rows_v.at[0], out_hbm.at[s, pl.ds(cid * w, w)])

    return conv_k


def _leaky(v):
    return jnp.where(v > 0, v, 0.01 * v)


def _pre_body(x_ref, w_ref, degt_ref, g_ref, dinv_ref):
    deg = jnp.sum(degt_ref[...], axis=1, keepdims=True) + 1.0
    dinv = lax.rsqrt(deg)
    h = jnp.dot(x_ref[...], w_ref[...], preferred_element_type=jnp.float32)
    g_ref[...] = h * dinv
    dinv_ref[...] = dinv


def _tc_pre(xp, w1, degt):
    npad, d = xp.shape
    dout = w1.shape[1]
    r = 1280
    rp = r * dout // 128
    return pl.pallas_call(
        _pre_body,
        grid=(npad // r,),
        in_specs=[
            pl.BlockSpec((r, d), lambda i: (i, 0)),
            pl.BlockSpec((d, dout), lambda i: (0, 0)),
            pl.BlockSpec((r, _NT), lambda i: (i, 0)),
        ],
        out_specs=[
            pl.BlockSpec((r, dout), lambda i: (i, 0)),
            pl.BlockSpec((r, 1), lambda i: (i, 0)),
        ],
        out_shape=[
            jax.ShapeDtypeStruct((npad, dout), jnp.float32),
            jax.ShapeDtypeStruct((npad, 1), jnp.float32),
        ],
    )(xp, w1, degt)


def _mid_body(din, acc_ref, g1_ref, dinv_ref, b1_ref, w2_ref, out_ref):
    a = acc_ref[:, :din] + acc_ref[:, din:2 * din]
    s = (a + g1_ref[...]) * dinv_ref[...] + b1_ref[...]
    h = _leaky(s)
    out_ref[...] = (
        jnp.dot(h, w2_ref[...], preferred_element_type=jnp.float32) * dinv_ref[...]
    )


def _tc_mid(acc, g1, dinv, b1, w2, npad):
    din = b1.shape[1]
    dout = w2.shape[1]
    r = 1280
    return pl.pallas_call(
        functools.partial(_mid_body, din),
        grid=(npad // r,),
        in_specs=[
            pl.BlockSpec((r, 128), lambda i: (i, 0)),
            pl.BlockSpec((r, din), lambda i: (i, 0)),
            pl.BlockSpec((r, 1), lambda i: (i, 0)),
            pl.BlockSpec((1, din), lambda i: (0, 0)),
            pl.BlockSpec((din, dout), lambda i: (0, 0)),
        ],
        out_specs=pl.BlockSpec((r, dout), lambda i: (i, 0)),
        out_shape=jax.ShapeDtypeStruct((npad, dout), jnp.float32),
    )(acc, g1, dinv, b1, w2)


def _fin_body(din, acc_ref, g2_ref, dinv_ref, b2_ref, encw_ref, encb_ref,
              dw1_ref, db1_ref, dw2_ref, db2_ref, lat_ref, rec_ref):
    a = acc_ref[:, :din] + acc_ref[:, din:2 * din]
    h = _leaky((a + g2_ref[...]) * dinv_ref[...] + b2_ref[...])
    z = jnp.dot(h, encw_ref[...], preferred_element_type=jnp.float32) + encb_ref[...]
    z = z - jnp.max(z, axis=1, keepdims=True)
    e = jnp.exp(z)
    latent = e / jnp.sum(e, axis=1, keepdims=True)
    lat_ref[...] = latent
    d = jnp.maximum(
        jnp.dot(latent, dw1_ref[...], preferred_element_type=jnp.float32)
        + db1_ref[...],
        0.0,
    )
    rec_ref[...] = (
        jnp.dot(d, dw2_ref[...], preferred_element_type=jnp.float32) + db2_ref[...]
    )


def _tc_fin(n, acc, g2, dinv, b2, encw, encb, dw1, db1, dw2, db2):
    din = b2.shape[1]
    dl = encw.shape[1]
    dmid = dw1.shape[1]
    dout = dw2.shape[1]
    r = 2000
    full = lambda shape: pl.BlockSpec(shape, lambda i: tuple(0 for _ in shape))
    return pl.pallas_call(
        functools.partial(_fin_body, din),
        grid=(n // r,),
        in_specs=[
            pl.BlockSpec((r, 128), lambda i: (i, 0)),
            pl.BlockSpec((r, din), lambda i: (i, 0)),
            pl.BlockSpec((r, 1), lambda i: (i, 0)),
            full((1, din)),
            full((din, dl)),
            full((1, dl)),
            full((dl, dmid)),
            full((1, dmid)),
            full((dmid, dout)),
            full((1, dout)),
        ],
        out_specs=[
            pl.BlockSpec((r, dl), lambda i: (i, 0)),
            pl.BlockSpec((r, dout), lambda i: (i, 0)),
        ],
        out_shape=[
            jax.ShapeDtypeStruct((n, dl), jnp.float32),
            jax.ShapeDtypeStruct((n, dout), jnp.float32),
        ],
    )(acc, g2, dinv, b2, encw, encb, dw1, db1, dw2, db2)


def kernel(x, edge_index, W1, b1, W2, b2, enc_W, enc_b, dec_W1, dec_b1,
           dec_W2, dec_b2):
    n, _ = x.shape
    e = edge_index.shape[1]
    npad = -(-(n + 1) // (_NS * _STRIPE)) * (_NS * _STRIPE)
    ch = -(-e // (_NT * _CHUNK))
    ep = _NT * _CHUNK * ch

    ei = edge_index.astype(jnp.int32)
    padv = n + jnp.arange(ep - e, dtype=jnp.int32) % (npad - n)
    srcp = jnp.concatenate([ei[0], padv]).reshape(_NT, ch, _CHUNK)
    dstp = jnp.concatenate([ei[1], padv]).reshape(_NT, ch, _CHUNK)
    xp = jnp.pad(x, ((0, npad - n), (0, 0)))

    degp = _make_deg_kernel(ch, npad)(dstp)
    g1 , dinv = _tc_pre(xp, W1, degp.T)
    acc1 = _make_conv_kernel(ch, npad, W1.shape[1])(g1, srcp, dstp)
    g2 = _tc_mid(acc1, g1, dinv, b1.reshape(1, -1), W2, npad)
    acc2 = _make_conv_kernel(ch, npad, W2.shape[1])(g2, srcp, dstp)
    latent, recon = _tc_fin(
        n, acc2, g2, dinv, b2.reshape(1, -1), enc_W,
        enc_b.reshape(1, -1), dec_W1, dec_b1.reshape(1, -1), dec_W2,
        dec_b2.reshape(1, -1),
    )
    return latent, recon


# speedup vs baseline: 1.2278x; 1.2278x over previous
import functools

import jax
import jax.numpy as jnp
from jax import lax
from jax.experimental import pallas as pl
from jax.experimental.pallas import tpu as pltpu
from jax.experimental.pallas import tpu_sc as plsc

_NC = 2
_NS = 16
_NT = _NC * _NS
_CHUNK = 128
_STRIPE = 128


def _sc_mesh():
    return plsc.VectorSubcoreMesh(core_axis_name="c", subcore_axis_name="s")


@functools.cache
def _make_deg_kernel(ch, npad):
    @functools.partial(
        pl.kernel,
        mesh=_sc_mesh(),
        compiler_params=pltpu.CompilerParams(needs_layout_passes=False, use_tc_tiling_on_sc=False),
        out_type=jax.ShapeDtypeStruct((_NT, npad), jnp.float32),
        scratch_types=[
            pltpu.VMEM((ch, _CHUNK), jnp.int32),
            pltpu.VMEM((npad,), jnp.float32),
        ],
    )
    def deg_k(dst_hbm, out_hbm, idx_v, deg_v):
        cid = lax.axis_index("c")
        sid = lax.axis_index("s")
        wid = cid * _NS + sid
        z16 = jnp.zeros((16,), jnp.float32)
        ones16 = jnp.ones((16,), jnp.float32)

        def zero_body(i, carry):
            deg_v[pl.ds(i * 16, 16)] = z16
            return carry

        lax.fori_loop(0, npad // 16, zero_body, 0)
        pltpu.sync_copy(dst_hbm.at[wid], idx_v)

        def acc_body(r, carry):
            for c in range(_CHUNK // 16):
                idx = idx_v[r, pl.ds(c * 16, 16)]
                plsc.addupdate_scatter(deg_v, [idx], ones16)
            return carry

        lax.fori_loop(0, ch, acc_body, 0)
        pltpu.sync_copy(deg_v, out_hbm.at[wid])

    return deg_k


@functools.cache
def _make_conv_kernel(ch, npad, w):
    rows_per_tile = npad // _NS
    nstripe = rows_per_tile // _STRIPE

    @functools.partial(
        pl.kernel,
        mesh=_sc_mesh(),
        compiler_params=pltpu.CompilerParams(needs_layout_passes=False, use_tc_tiling_on_sc=False),
        out_type=jax.ShapeDtypeStruct((npad, 128), jnp.float32),
        scratch_types=[
            pltpu.VMEM((ch, _CHUNK), jnp.int32),
            pltpu.VMEM((ch, _CHUNK), jnp.int32),
            pltpu.VMEM((4, _CHUNK, w), jnp.float32),
            pltpu.VMEM_SHARED((npad, w), jnp.float32),
            pltpu.SemaphoreType.DMA,
            pltpu.SemaphoreType.DMA,
            pltpu.SemaphoreType.DMA,
            pltpu.SemaphoreType.DMA,
        ],
    )
    def conv_k(g_hbm, src_hbm, dst_hbm, out_hbm, src_v, dst_v, rows_v, acc_sh,
               sem0, sem1, sem2, sem3):
        cid = lax.axis_index("c")
        sid = lax.axis_index("s")
        wid = cid * _NS + sid
        sems = (sem0, sem1, sem2, sem3)
        z16 = jnp.zeros((16,), jnp.float32)

        def zero_body(r, carry):
            for c in range(w // 16):
                rows_v[0, r, pl.ds(c * 16, 16)] = z16
            return carry

        lax.fori_loop(0, _CHUNK, zero_body, 0)
        base = sid * rows_per_tile
        for k in range(nstripe):
            pltpu.sync_copy(
                rows_v.at[0], acc_sh.at[pl.ds(base + k * _STRIPE, _STRIPE)]
            )
        pltpu.sync_copy(src_hbm.at[wid], src_v)
        pltpu.sync_copy(dst_hbm.at[wid], dst_v)
        plsc.subcore_barrier()

        def start_gather(j, b):
            pltpu.async_copy(g_hbm.at[src_v.at[j]], rows_v.at[b], sems[b])

        def finish(j, b):
            pltpu.make_async_copy(
                g_hbm.at[src_v.at[j]], rows_v.at[b], sems[b]
            ).wait()
            pltpu.sync_copy(rows_v.at[b], acc_sh.at[dst_v.at[j]], add=True)

        depth = 4
        pro = min(depth - 1, ch)
        for j in range(pro):
            start_gather(j, j % depth)
        nq = (ch - pro) // depth

        def quad_body(q, carry):
            j0 = depth * q
            for k in range(depth):
                start_gather(j0 + pro + k, (pro + k) % depth)
                finish(j0 + k, k)
            return carry

        lax.fori_loop(0, nq, quad_body, 0)
        s = pro + depth * nq
        for j in range(depth * nq, ch):
            finish(j, j % depth)
            if s < ch:
                start_gather(s, s % depth)
                s += 1
        plsc.subcore_barrier()
        for k in range(nstripe):
            s = pl.ds(base + k * _STRIPE, _STRIPE)
            pltpu.sync_copy(acc_sh.at[s], rows_v.at[0])
            pltpu.sync_copy(rows_v.at[0], out_hbm.at[s, pl.ds(cid * w, w)])

    return conv_k


def _leaky(v):
    return jnp.where(v > 0, v, 0.01 * v)


def _pre_body(x_ref, w_ref, degt_ref, g_ref, dinv_ref):
    deg = jnp.sum(degt_ref[...], axis=1, keepdims=True) + 1.0
    dinv = lax.rsqrt(deg)
    h = jnp.dot(x_ref[...], w_ref[...], preferred_element_type=jnp.float32)
    g_ref[...] = h * dinv
    dinv_ref[...] = dinv


def _tc_pre(xp, w1, degt):
    npad, d = xp.shape
    dout = w1.shape[1]
    r = 1280
    rp = r * dout // 128
    return pl.pallas_call(
        _pre_body,
        grid=(npad // r,),
        in_specs=[
            pl.BlockSpec((r, d), lambda i: (i, 0)),
            pl.BlockSpec((d, dout), lambda i: (0, 0)),
            pl.BlockSpec((r, _NT), lambda i: (i, 0)),
        ],
        out_specs=[
            pl.BlockSpec((r, dout), lambda i: (i, 0)),
            pl.BlockSpec((r, 1), lambda i: (i, 0)),
        ],
        out_shape=[
            jax.ShapeDtypeStruct((npad, dout), jnp.float32),
            jax.ShapeDtypeStruct((npad, 1), jnp.float32),
        ],
    )(xp, w1, degt)


def _mid_body(din, acc_ref, g1_ref, dinv_ref, b1_ref, w2_ref, out_ref):
    a = acc_ref[:, :din] + acc_ref[:, din:2 * din]
    s = (a + g1_ref[...]) * dinv_ref[...] + b1_ref[...]
    h = _leaky(s)
    out_ref[...] = (
        jnp.dot(h, w2_ref[...], preferred_element_type=jnp.float32) * dinv_ref[...]
    )


def _tc_mid(acc, g1, dinv, b1, w2, npad):
    din = b1.shape[1]
    dout = w2.shape[1]
    r = 1280
    return pl.pallas_call(
        functools.partial(_mid_body, din),
        grid=(npad // r,),
        in_specs=[
            pl.BlockSpec((r, 128), lambda i: (i, 0)),
            pl.BlockSpec((r, din), lambda i: (i, 0)),
            pl.BlockSpec((r, 1), lambda i: (i, 0)),
            pl.BlockSpec((1, din), lambda i: (0, 0)),
            pl.BlockSpec((din, dout), lambda i: (0, 0)),
        ],
        out_specs=pl.BlockSpec((r, dout), lambda i: (i, 0)),
        out_shape=jax.ShapeDtypeStruct((npad, dout), jnp.float32),
    )(acc, g1, dinv, b1, w2)


def _fin_body(din, acc_ref, g2_ref, dinv_ref, b2_ref, encw_ref, encb_ref,
              dw1_ref, db1_ref, dw2_ref, db2_ref, lat_ref, rec_ref):
    a = acc_ref[:, :din] + acc_ref[:, din:2 * din]
    h = _leaky((a + g2_ref[...]) * dinv_ref[...] + b2_ref[...])
    z = jnp.dot(h, encw_ref[...], preferred_element_type=jnp.float32) + encb_ref[...]
    z = z - jnp.max(z, axis=1, keepdims=True)
    e = jnp.exp(z)
    latent = e / jnp.sum(e, axis=1, keepdims=True)
    lat_ref[...] = latent
    d = jnp.maximum(
        jnp.dot(latent, dw1_ref[...], preferred_element_type=jnp.float32)
        + db1_ref[...],
        0.0,
    )
    rec_ref[...] = (
        jnp.dot(d, dw2_ref[...], preferred_element_type=jnp.float32) + db2_ref[...]
    )


def _tc_fin(n, acc, g2, dinv, b2, encw, encb, dw1, db1, dw2, db2):
    din = b2.shape[1]
    dl = encw.shape[1]
    dmid = dw1.shape[1]
    dout = dw2.shape[1]
    r = 2000
    full = lambda shape: pl.BlockSpec(shape, lambda i: tuple(0 for _ in shape))
    return pl.pallas_call(
        functools.partial(_fin_body, din),
        grid=(n // r,),
        in_specs=[
            pl.BlockSpec((r, 128), lambda i: (i, 0)),
            pl.BlockSpec((r, din), lambda i: (i, 0)),
            pl.BlockSpec((r, 1), lambda i: (i, 0)),
            full((1, din)),
            full((din, dl)),
            full((1, dl)),
            full((dl, dmid)),
            full((1, dmid)),
            full((dmid, dout)),
            full((1, dout)),
        ],
        out_specs=[
            pl.BlockSpec((r, dl), lambda i: (i, 0)),
            pl.BlockSpec((r, dout), lambda i: (i, 0)),
        ],
        out_shape=[
            jax.ShapeDtypeStruct((n, dl), jnp.float32),
            jax.ShapeDtypeStruct((n, dout), jnp.float32),
        ],
    )(acc, g2, dinv, b2, encw, encb, dw1, db1, dw2, db2)


def kernel(x, edge_index, W1, b1, W2, b2, enc_W, enc_b, dec_W1, dec_b1,
           dec_W2, dec_b2):
    n, _ = x.shape
    e = edge_index.shape[1]
    npad = -(-(n + 1) // (_NS * _STRIPE)) * (_NS * _STRIPE)
    ch = -(-e // (_NT * _CHUNK))
    ep = _NT * _CHUNK * ch

    ei = edge_index.astype(jnp.int32)
    padv = n + jnp.arange(ep - e, dtype=jnp.int32) % (npad - n)
    srcp = jnp.concatenate([ei[0], padv]).reshape(_NT, ch, _CHUNK)
    dstp = jnp.concatenate([ei[1], padv]).reshape(_NT, ch, _CHUNK)
    xp = jnp.pad(x, ((0, npad - n), (0, 0)))

    degp = _make_deg_kernel(ch, npad)(dstp)
    g1 , dinv = _tc_pre(xp, W1, degp.T)
    acc1 = _make_conv_kernel(ch, npad, W1.shape[1])(g1, srcp, dstp)
    g2 = _tc_mid(acc1, g1, dinv, b1.reshape(1, -1), W2, npad)
    acc2 = _make_conv_kernel(ch, npad, W2.shape[1])(g2, srcp, dstp)
    latent, recon = _tc_fin(
        n, acc2, g2, dinv, b2.reshape(1, -1), enc_W,
        enc_b.reshape(1, -1), dec_W1, dec_b1.reshape(1, -1), dec_W2,
        dec_b2.reshape(1, -1),
    )
    return latent, recon


# speedup vs baseline: 1.2397x; 1.0097x over previous
import functools

import jax
import jax.numpy as jnp
from jax import lax
from jax.experimental import pallas as pl
from jax.experimental.pallas import tpu as pltpu
from jax.experimental.pallas import tpu_sc as plsc

_NC = 2
_NS = 16
_NT = _NC * _NS
_CHUNK = 128
_STRIPE = 128


def _sc_mesh():
    return plsc.VectorSubcoreMesh(core_axis_name="c", subcore_axis_name="s")


@functools.cache
def _make_deg_kernel(ch, npad):
    @functools.partial(
        pl.kernel,
        mesh=_sc_mesh(),
        compiler_params=pltpu.CompilerParams(needs_layout_passes=False, use_tc_tiling_on_sc=False),
        out_type=jax.ShapeDtypeStruct((_NT, npad), jnp.float32),
        scratch_types=[
            pltpu.VMEM((ch, _CHUNK), jnp.int32),
            pltpu.VMEM((npad,), jnp.float32),
        ],
    )
    def deg_k(dst_hbm, out_hbm, idx_v, deg_v):
        cid = lax.axis_index("c")
        sid = lax.axis_index("s")
        wid = cid * _NS + sid
        z16 = jnp.zeros((16,), jnp.float32)
        ones16 = jnp.ones((16,), jnp.float32)

        def zero_body(i, carry):
            deg_v[pl.ds(i * 16, 16)] = z16
            return carry

        lax.fori_loop(0, npad // 16, zero_body, 0)
        pltpu.sync_copy(dst_hbm.at[wid], idx_v)

        def acc_body(r, carry):
            for c in range(_CHUNK // 16):
                idx = idx_v[r, pl.ds(c * 16, 16)]
                plsc.addupdate_scatter(deg_v, [idx], ones16)
            return carry

        lax.fori_loop(0, ch, acc_body, 0)
        pltpu.sync_copy(deg_v, out_hbm.at[wid])

    return deg_k


@functools.cache
def _make_conv_kernel(ch, npad, w):
    rows_per_tile = npad // _NS
    nstripe = rows_per_tile // _STRIPE

    @functools.partial(
        pl.kernel,
        mesh=_sc_mesh(),
        compiler_params=pltpu.CompilerParams(needs_layout_passes=False, use_tc_tiling_on_sc=False),
        out_type=jax.ShapeDtypeStruct((npad, 128), jnp.float32),
        scratch_types=[
            pltpu.VMEM((ch, _CHUNK), jnp.int32),
            pltpu.VMEM((ch, _CHUNK), jnp.int32),
            pltpu.VMEM((8, _CHUNK, w), jnp.float32),
            pltpu.VMEM_SHARED((npad, w), jnp.float32),
            pltpu.SemaphoreType.DMA,
            pltpu.SemaphoreType.DMA,
            pltpu.SemaphoreType.DMA,
            pltpu.SemaphoreType.DMA,
            pltpu.SemaphoreType.DMA,
            pltpu.SemaphoreType.DMA,
            pltpu.SemaphoreType.DMA,
            pltpu.SemaphoreType.DMA,
        ],
    )
    def conv_k(g_hbm, src_hbm, dst_hbm, out_hbm, src_v, dst_v, rows_v, acc_sh,
               sem0, sem1, sem2, sem3, sem4, sem5, sem6, sem7):
        cid = lax.axis_index("c")
        sid = lax.axis_index("s")
        wid = cid * _NS + sid
        sems = (sem0, sem1, sem2, sem3, sem4, sem5, sem6, sem7)
        z16 = jnp.zeros((16,), jnp.float32)

        def zero_body(r, carry):
            for c in range(w // 16):
                rows_v[0, r, pl.ds(c * 16, 16)] = z16
            return carry

        lax.fori_loop(0, _CHUNK, zero_body, 0)
        base = sid * rows_per_tile
        for k in range(nstripe):
            pltpu.sync_copy(
                rows_v.at[0], acc_sh.at[pl.ds(base + k * _STRIPE, _STRIPE)]
            )
        pltpu.sync_copy(src_hbm.at[wid], src_v)
        pltpu.sync_copy(dst_hbm.at[wid], dst_v)
        plsc.subcore_barrier()

        def start_gather(j, b):
            pltpu.async_copy(g_hbm.at[src_v.at[j]], rows_v.at[b], sems[b])

        def finish(j, b):
            pltpu.make_async_copy(
                g_hbm.at[src_v.at[j]], rows_v.at[b], sems[b]
            ).wait()
            pltpu.sync_copy(rows_v.at[b], acc_sh.at[dst_v.at[j]], add=True)

        depth = 8
        pro = min(depth - 1, ch)
        for j in range(pro):
            start_gather(j, j % depth)
        nq = (ch - pro) // depth

        def quad_body(q, carry):
            j0 = depth * q
            for k in range(depth):
                start_gather(j0 + pro + k, (pro + k) % depth)
                finish(j0 + k, k)
            return carry

        lax.fori_loop(0, nq, quad_body, 0)
        s = pro + depth * nq
        for j in range(depth * nq, ch):
            finish(j, j % depth)
            if s < ch:
                start_gather(s, s % depth)
                s += 1
        plsc.subcore_barrier()
        for k in range(nstripe):
            s = pl.ds(base + k * _STRIPE, _STRIPE)
            pltpu.sync_copy(acc_sh.at[s], rows_v.at[0])
            pltpu.sync_copy(rows_v.at[0], out_hbm.at[s, pl.ds(cid * w, w)])

    return conv_k


def _leaky(v):
    return jnp.where(v > 0, v, 0.01 * v)


def _pre_body(x_ref, w_ref, degt_ref, g_ref, dinv_ref):
    deg = jnp.sum(degt_ref[...], axis=1, keepdims=True) + 1.0
    dinv = lax.rsqrt(deg)
    h = jnp.dot(x_ref[...], w_ref[...], preferred_element_type=jnp.float32)
    g_ref[...] = h * dinv
    dinv_ref[...] = dinv


def _tc_pre(xp, w1, degt):
    npad, d = xp.shape
    dout = w1.shape[1]
    r = 1280
    rp = r * dout // 128
    return pl.pallas_call(
        _pre_body,
        grid=(npad // r,),
        in_specs=[
            pl.BlockSpec((r, d), lambda i: (i, 0)),
            pl.BlockSpec((d, dout), lambda i: (0, 0)),
            pl.BlockSpec((r, _NT), lambda i: (i, 0)),
        ],
        out_specs=[
            pl.BlockSpec((r, dout), lambda i: (i, 0)),
            pl.BlockSpec((r, 1), lambda i: (i, 0)),
        ],
        out_shape=[
            jax.ShapeDtypeStruct((npad, dout), jnp.float32),
            jax.ShapeDtypeStruct((npad, 1), jnp.float32),
        ],
    )(xp, w1, degt)


def _mid_body(din, acc_ref, g1_ref, dinv_ref, b1_ref, w2_ref, out_ref):
    a = acc_ref[:, :din] + acc_ref[:, din:2 * din]
    s = (a + g1_ref[...]) * dinv_ref[...] + b1_ref[...]
    h = _leaky(s)
    out_ref[...] = (
        jnp.dot(h, w2_ref[...], preferred_element_type=jnp.float32) * dinv_ref[...]
    )


def _tc_mid(acc, g1, dinv, b1, w2, npad):
    din = b1.shape[1]
    dout = w2.shape[1]
    r = 1280
    return pl.pallas_call(
        functools.partial(_mid_body, din),
        grid=(npad // r,),
        in_specs=[
            pl.BlockSpec((r, 128), lambda i: (i, 0)),
            pl.BlockSpec((r, din), lambda i: (i, 0)),
            pl.BlockSpec((r, 1), lambda i: (i, 0)),
            pl.BlockSpec((1, din), lambda i: (0, 0)),
            pl.BlockSpec((din, dout), lambda i: (0, 0)),
        ],
        out_specs=pl.BlockSpec((r, dout), lambda i: (i, 0)),
        out_shape=jax.ShapeDtypeStruct((npad, dout), jnp.float32),
    )(acc, g1, dinv, b1, w2)


def _fin_body(din, acc_ref, g2_ref, dinv_ref, b2_ref, encw_ref, encb_ref,
              dw1_ref, db1_ref, dw2_ref, db2_ref, lat_ref, rec_ref):
    a = acc_ref[:, :din] + acc_ref[:, din:2 * din]
    h = _leaky((a + g2_ref[...]) * dinv_ref[...] + b2_ref[...])
    z = jnp.dot(h, encw_ref[...], preferred_element_type=jnp.float32) + encb_ref[...]
    z = z - jnp.max(z, axis=1, keepdims=True)
    e = jnp.exp(z)
    latent = e / jnp.sum(e, axis=1, keepdims=True)
    lat_ref[...] = latent
    d = jnp.maximum(
        jnp.dot(latent, dw1_ref[...], preferred_element_type=jnp.float32)
        + db1_ref[...],
        0.0,
    )
    rec_ref[...] = (
        jnp.dot(d, dw2_ref[...], preferred_element_type=jnp.float32) + db2_ref[...]
    )


def _tc_fin(n, acc, g2, dinv, b2, encw, encb, dw1, db1, dw2, db2):
    din = b2.shape[1]
    dl = encw.shape[1]
    dmid = dw1.shape[1]
    dout = dw2.shape[1]
    r = 2000
    full = lambda shape: pl.BlockSpec(shape, lambda i: tuple(0 for _ in shape))
    return pl.pallas_call(
        functools.partial(_fin_body, din),
        grid=(n // r,),
        in_specs=[
            pl.BlockSpec((r, 128), lambda i: (i, 0)),
            pl.BlockSpec((r, din), lambda i: (i, 0)),
            pl.BlockSpec((r, 1), lambda i: (i, 0)),
            full((1, din)),
            full((din, dl)),
            full((1, dl)),
            full((dl, dmid)),
            full((1, dmid)),
            full((dmid, dout)),
            full((1, dout)),
        ],
        out_specs=[
            pl.BlockSpec((r, dl), lambda i: (i, 0)),
            pl.BlockSpec((r, dout), lambda i: (i, 0)),
        ],
        out_shape=[
            jax.ShapeDtypeStruct((n, dl), jnp.float32),
            jax.ShapeDtypeStruct((n, dout), jnp.float32),
        ],
    )(acc, g2, dinv, b2, encw, encb, dw1, db1, dw2, db2)


def kernel(x, edge_index, W1, b1, W2, b2, enc_W, enc_b, dec_W1, dec_b1,
           dec_W2, dec_b2):
    n, _ = x.shape
    e = edge_index.shape[1]
    npad = -(-(n + 1) // (_NS * _STRIPE)) * (_NS * _STRIPE)
    ch = -(-e // (_NT * _CHUNK))
    ep = _NT * _CHUNK * ch

    ei = edge_index.astype(jnp.int32)
    padv = n + jnp.arange(ep - e, dtype=jnp.int32) % (npad - n)
    srcp = jnp.concatenate([ei[0], padv]).reshape(_NT, ch, _CHUNK)
    dstp = jnp.concatenate([ei[1], padv]).reshape(_NT, ch, _CHUNK)
    xp = jnp.pad(x, ((0, npad - n), (0, 0)))

    degp = _make_deg_kernel(ch, npad)(dstp)
    g1 , dinv = _tc_pre(xp, W1, degp.T)
    acc1 = _make_conv_kernel(ch, npad, W1.shape[1])(g1, srcp, dstp)
    g2 = _tc_mid(acc1, g1, dinv, b1.reshape(1, -1), W2, npad)
    acc2 = _make_conv_kernel(ch, npad, W2.shape[1])(g2, srcp, dstp)
    latent, recon = _tc_fin(
        n, acc2, g2, dinv, b2.reshape(1, -1), enc_W,
        enc_b.reshape(1, -1), dec_W1, dec_b1.reshape(1, -1), dec_W2,
        dec_b2.reshape(1, -1),
    )
    return latent, recon
